# baseline (speedup 1.0000x reference)
import functools

import jax
import jax.numpy as jnp
from jax import lax
from jax.experimental import pallas as pl
from jax.experimental.pallas import tpu as pltpu
from jax.experimental.pallas import tpu_sc as plsc

B, L, D, K = 4, 8192, 768, 1024
NW = 32
RPW = (B * K) // NW


def _body(ta, tb, st, ia, ib, ica, icb, out_s, out_t,
          i0, i1, i2, i3, rows_v, isem, gsem, wsem):
    wid = lax.axis_index("s") * 2 + lax.axis_index("c")
    flat_base = pl.multiple_of(wid * RPW, RPW)
    b = flat_base // K
    k_base = flat_base - b * K
    out_bb = b * (2 * K)

    jobs = (
        (ta, ia, out_t, 0, i0),
        (tb, ib, out_t, K, i1),
        (st, ica, out_s, 0, i2),
        (st, icb, out_s, K, i3),
    )
    icopies = [
        pltpu.async_copy(iref.at[b, pl.ds(k_base, RPW)], iv, isem)
        for (_, iref, _, _, iv) in jobs
    ]
    write = None
    for (tab, _, oref, joff, iv), ic in zip(jobs, icopies):
        ic.wait()
        if write is not None:
            write.wait()
        g = pltpu.async_copy(tab.at[b].at[iv], rows_v, gsem)
        g.wait()
        out_base = pl.multiple_of(out_bb + joff + k_base, RPW)
        write = pltpu.async_copy(rows_v, oref.at[pl.ds(out_base, RPW)], wsem)
    write.wait()


@functools.partial(
    pl.kernel,
    mesh=plsc.VectorSubcoreMesh(core_axis_name="c", subcore_axis_name="s"),
    out_type=[
        jax.ShapeDtypeStruct((B * 2 * K, D), jnp.float32),
        jax.ShapeDtypeStruct((B * 2 * K, D), jnp.float32),
    ],
    scratch_types=[
        pltpu.VMEM((RPW,), jnp.int32),
        pltpu.VMEM((RPW,), jnp.int32),
        pltpu.VMEM((RPW,), jnp.int32),
        pltpu.VMEM((RPW,), jnp.int32),
        pltpu.VMEM((RPW, D), jnp.float32),
        pltpu.SemaphoreType.DMA,
        pltpu.SemaphoreType.DMA,
        pltpu.SemaphoreType.DMA,
    ],
)
def _gather(ta, tb, st, ia, ib, ica, icb, out_s, out_t,
            i0, i1, i2, i3, rows_v, isem, gsem, wsem):
    _body(ta, tb, st, ia, ib, ica, icb, out_s, out_t,
          i0, i1, i2, i3, rows_v, isem, gsem, wsem)


def kernel(teacher_logits_a, teacher_logits_b, student_results,
           span_a_selected_index, span_b_selected_index,
           span_c_a_selected_index, span_c_b_selected_index):
    out_s, out_t = _gather(
        teacher_logits_a, teacher_logits_b, student_results,
        span_a_selected_index.astype(jnp.int32),
        span_b_selected_index.astype(jnp.int32),
        span_c_a_selected_index.astype(jnp.int32),
        span_c_b_selected_index.astype(jnp.int32))
    return (out_s.reshape(B, 2 * K, D), out_t.reshape(B, 2 * K, D))
